# Initial kernel scaffold; baseline (speedup 1.0000x reference)
import jax
import jax.numpy as jnp
from jax.experimental import pallas as pl


def kernel(confidence, predicted_locations, labels, gt_locations):
    raise NotImplementedError("write your pallas kernel here")


# speedup vs baseline: 1.0080x; 1.0080x over previous
import functools

import jax
import jax.numpy as jnp
from jax import lax
from jax.experimental import pallas as pl
from jax.experimental.pallas import tpu as pltpu
from jax.experimental.pallas import tpu_sc as plsc

B, P, C = 32, 8732, 81
PP = 8736
TR = 4736
NBLK = (B * P) // TR
GRP = PP // 16


def _tc_body(x_ref, lab_ref, pred_ref, gt_ref, ce_ref, bb_ref):
    x = x_ref[...]
    lab = lab_ref[...]
    m = jnp.max(x, axis=1, keepdims=True)
    s = jnp.sum(jnp.exp(x - m), axis=1, keepdims=True)
    lse = m + jnp.log(s)
    col = lax.broadcasted_iota(jnp.int32, (TR, C), 1)
    cl = jnp.sum(jnp.where(col == lab, x, 0.0), axis=1, keepdims=True)
    ce_ref[...] = lse - cl

    d = pred_ref[...] - gt_ref[...]
    ad = jnp.abs(d)
    sl1 = jnp.where(ad < 1.0, 0.5 * d * d, ad - 0.5)
    posf = (lab > 0).astype(jnp.float32)
    bb = jnp.sum(sl1 * posf)

    @pl.when(pl.program_id(0) == 0)
    def _init():
        bb_ref[...] = jnp.zeros_like(bb_ref)

    lane = lax.broadcasted_iota(jnp.int32, (1, 16), 1)
    bb_ref[...] += jnp.where(lane == 0, bb, 0.0)


def _tc_stage(conf2, lab2, pred2, gt2):
    return pl.pallas_call(
        _tc_body,
        grid=(NBLK,),
        in_specs=[
            pl.BlockSpec((TR, C), lambda i: (i, 0)),
            pl.BlockSpec((TR, 1), lambda i: (i, 0)),
            pl.BlockSpec((TR, 4), lambda i: (i, 0)),
            pl.BlockSpec((TR, 4), lambda i: (i, 0)),
        ],
        out_specs=[
            pl.BlockSpec((TR, 1), lambda i: (i, 0)),
            pl.BlockSpec((1, 16), lambda i: (0, 0)),
        ],
        out_shape=[
            jax.ShapeDtypeStruct((B * P, 1), jnp.float32),
            jax.ShapeDtypeStruct((1, 16), jnp.float32),
        ],
    )(conf2, lab2, pred2, gt2)


_IOTA16 = lambda: lax.iota(jnp.int32, 16)


def _hist_zero(hcnt, hsum, nb):
    zi = jnp.zeros((16,), jnp.int32)
    zf = jnp.zeros((16,), jnp.float32)

    def zbody(i, _):
        hcnt[pl.ds(i * 16, 16)] = zi
        hsum[pl.ds(i * 16, 16)] = zf
        return 0

    lax.fori_loop(0, nb // 16, zbody, 0)


def _hist_fill(ce_v, lab_v, hcnt, hsum, shift, dmask, pshift, pref):
    ones = jnp.ones((16,), jnp.int32)

    def body(i, _):
        off = i * 16
        v = ce_v[pl.ds(off, 16)]
        l = lab_v[pl.ds(off, 16)]
        u = plsc.bitcast(v, jnp.int32)
        msk = (l == 0) & ((u >> pshift) == pref)
        d = (u >> shift) & dmask
        plsc.addupdate_scatter(hcnt, [d], ones, mask=msk)
        plsc.addupdate_scatter(hsum, [d], v, mask=msk)
        return 0

    lax.fori_loop(0, GRP, body, 0)


def _hist_scan(hcnt, hsum, nb, K, G0, S0):
    iota16 = _IOTA16()
    nchunk = nb // 16

    def sbody(jj, st):
        G, S, b, cb, found = st
        j = nchunk - 1 - jj
        cv = hcnt[pl.ds(j * 16, 16)]
        sv = hsum[pl.ds(j * 16, 16)]
        rcv = lax.rev(cv, (0,))
        rsv = lax.rev(sv, (0,))
        ccum = plsc.cumsum(rcv)
        scum = plsc.cumsum(rsv)
        cross = (G + ccum) >= K
        r = jnp.min(jnp.where(cross, iota16, 16))
        hit = r < 16
        oh = iota16 == r
        cR = jnp.sum(jnp.where(oh, ccum, 0))
        cntR = jnp.sum(jnp.where(oh, rcv, 0))
        sR = jnp.sum(jnp.where(oh, scum, 0.0))
        sumR = jnp.sum(jnp.where(oh, rsv, 0.0))
        tot = jnp.sum(cv)
        stot = jnp.sum(sv)
        nf = found | hit
        G2 = jnp.where(found, G, jnp.where(hit, G + cR - cntR, G + tot))
        S2 = jnp.where(found, S, jnp.where(hit, S + sR - sumR, S + stot))
        b2 = jnp.where(found, b, jnp.where(hit, j * 16 + 15 - r, b))
        cb2 = jnp.where(found, cb, jnp.where(hit, cntR, cb))
        return G2, S2, b2, cb2, nf

    G, S, b, cb, _ = lax.fori_loop(
        0, nchunk, sbody, (G0, S0, jnp.int32(0), jnp.int32(0), False))
    return b, G, S, cb


def _sc_body(ce_hbm, lab_hbm, out_hbm, ce_v, lab_v, hcnt, hsum, outv):
    cax = lax.axis_index("c")
    sax = lax.axis_index("s")
    wid = sax * 2 + cax

    pltpu.sync_copy(ce_hbm.at[wid], ce_v)
    pltpu.sync_copy(lab_hbm.at[wid], lab_v)

    zi = jnp.zeros((16,), jnp.int32)
    zf = jnp.zeros((16,), jnp.float32)

    def p1(i, acc):
        npos, pce, nneg, nsum = acc
        off = i * 16
        v = ce_v[pl.ds(off, 16)]
        l = lab_v[pl.ds(off, 16)]
        pos = l > 0
        neg = l == 0
        return (npos + jnp.where(pos, 1, 0),
                pce + jnp.where(pos, v, 0.0),
                nneg + jnp.where(neg, 1, 0),
                nsum + jnp.where(neg, v, 0.0))

    npos_v, pce_v, nneg_v, nsum_v = lax.fori_loop(0, GRP, p1, (zi, zf, zi, zf))
    num_pos = jnp.sum(npos_v)
    n_neg = jnp.sum(nneg_v)
    pos_ce = jnp.sum(pce_v)
    neg_all = jnp.sum(nsum_v)
    K = num_pos * 3

    def all_branch():
        return neg_all

    def radix_branch():
        G = jnp.int32(0)
        S = jnp.float32(0.0)
        _hist_zero(hcnt, hsum, 2048)
        _hist_fill(ce_v, lab_v, hcnt, hsum, 20, 2047, 31, jnp.int32(0))
        b0, G, S, _ = _hist_scan(hcnt, hsum, 2048, K, G, S)
        _hist_zero(hcnt, hsum, 1024)
        _hist_fill(ce_v, lab_v, hcnt, hsum, 10, 1023, 20, b0)
        b1, G, S, _ = _hist_scan(hcnt, hsum, 1024, K, G, S)
        _hist_zero(hcnt, hsum, 1024)
        pref2 = (b0 << 10) | b1
        _hist_fill(ce_v, lab_v, hcnt, hsum, 0, 1023, 10, pref2)
        b2, G, S, T = _hist_scan(hcnt, hsum, 1024, K, G, S)
        vbits = (pref2 << 10) | b2
        V = lax.bitcast_convert_type(vbits, jnp.float32)
        taken = jnp.clip(K - G, 0, T)
        return S + jnp.where(taken > 0, taken.astype(jnp.float32) * V, 0.0)

    neg_sel = lax.cond(K >= n_neg, all_branch, radix_branch)
    cls_row = pos_ce + neg_sel

    iota16 = _IOTA16()
    outv[...] = jnp.where(iota16 == 0, cls_row,
                          jnp.where(iota16 == 1, num_pos.astype(jnp.float32),
                                    0.0))
    pltpu.sync_copy(outv, out_hbm.at[wid])


def _sc_stage(ce_pad, lab_pad):
    mesh = plsc.VectorSubcoreMesh(core_axis_name="c", subcore_axis_name="s",
                                  num_cores=2, num_subcores=16)
    f = functools.partial(
        pl.kernel,
        out_type=jax.ShapeDtypeStruct((B, 16), jnp.float32),
        mesh=mesh,
        compiler_params=pltpu.CompilerParams(needs_layout_passes=False),
        scratch_types=[
            pltpu.VMEM((PP,), jnp.float32),
            pltpu.VMEM((PP,), jnp.int32),
            pltpu.VMEM((2048,), jnp.int32),
            pltpu.VMEM((2048,), jnp.float32),
            pltpu.VMEM((16,), jnp.float32),
        ],
    )(_sc_body)
    return f(ce_pad, lab_pad)


def kernel(confidence, predicted_locations, labels, gt_locations):
    conf2 = confidence.reshape(B * P, C)
    lab2 = labels.reshape(B * P, 1)
    pred2 = predicted_locations.reshape(B * P, 4)
    gt2 = gt_locations.reshape(B * P, 4)

    ce_flat, bbvec = _tc_stage(conf2, lab2, pred2, gt2)

    ce_pad = jnp.pad(ce_flat.reshape(B, P), ((0, 0), (0, PP - P)))
    lab_pad = jnp.pad(labels, ((0, 0), (0, PP - P)), constant_values=-1)

    rows = _sc_stage(ce_pad, lab_pad)

    cls_total = jnp.sum(rows[:, 0])
    npos_total = jnp.sum(rows[:, 1])
    bbox_loss = bbvec[0, 0] / npos_total
    cls_loss = cls_total / npos_total
    return (bbox_loss, cls_loss)


# speedup vs baseline: 1.3356x; 1.3250x over previous
import functools

import jax
import jax.numpy as jnp
from jax import lax
from jax.experimental import pallas as pl
from jax.experimental.pallas import tpu as pltpu
from jax.experimental.pallas import tpu_sc as plsc

B, P, C = 32, 8732, 81
PP = 8960
TP = 4480
NPT = PP // TP
GRP = PP // 16


def _tc_body(x_ref, lab_ref, pred_ref, gt_ref, ce_ref, bb_ref):
    pt = pl.program_id(1)
    x = x_ref[0]
    lab = lab_ref[0]
    m = jnp.max(x, axis=1, keepdims=True)
    s = jnp.sum(jnp.exp(x - m), axis=1, keepdims=True)
    col = lax.broadcasted_iota(jnp.int32, (TP, C), 1)
    cl = jnp.sum(jnp.where(col == lab, x, 0.0), axis=1, keepdims=True)
    msc = jnp.concatenate([m, s, cl], axis=1)
    msc_t = msc.T
    lse_t = msc_t[0:1, :] + jnp.log(msc_t[1:2, :])
    ce_ref[0, 0, pl.ds(pt * TP, TP)] = (lse_t - msc_t[2:3, :])[0]

    row = lax.broadcasted_iota(jnp.int32, (TP, 1), 0)
    valid = (pt * TP + row) < P
    d = pred_ref[0] - gt_ref[0]
    ad = jnp.abs(d)
    sl1 = jnp.where(ad < 1.0, 0.5 * d * d, ad - 0.5)
    posf = jnp.where(valid & (lab > 0), 1.0, 0.0)
    bb = jnp.sum(sl1 * posf)

    @pl.when((pl.program_id(0) == 0) & (pt == 0))
    def _init():
        bb_ref[...] = jnp.zeros_like(bb_ref)

    lane = lax.broadcasted_iota(jnp.int32, (1, 16), 1)
    bb_ref[...] += jnp.where(lane == 0, bb, 0.0)


def _tc_stage(conf, lab3, pred, gt):
    return pl.pallas_call(
        _tc_body,
        grid=(B, NPT),
        in_specs=[
            pl.BlockSpec((1, TP, C), lambda b, p: (b, p, 0)),
            pl.BlockSpec((1, TP, 1), lambda b, p: (b, p, 0)),
            pl.BlockSpec((1, TP, 4), lambda b, p: (b, p, 0)),
            pl.BlockSpec((1, TP, 4), lambda b, p: (b, p, 0)),
        ],
        out_specs=[
            pl.BlockSpec((1, 1, PP), lambda b, p: (b, 0, 0)),
            pl.BlockSpec((1, 16), lambda b, p: (0, 0)),
        ],
        out_shape=[
            jax.ShapeDtypeStruct((B, 1, PP), jnp.float32),
            jax.ShapeDtypeStruct((1, 16), jnp.float32),
        ],
    )(conf, lab3, pred, gt)


_IOTA16 = lambda: lax.iota(jnp.int32, 16)


def _hist_zero(hcnt, hsum, nb):
    zi = jnp.zeros((16,), jnp.int32)
    zf = jnp.zeros((16,), jnp.float32)

    def zbody(i, _):
        hcnt[pl.ds(i * 16, 16)] = zi
        hsum[pl.ds(i * 16, 16)] = zf
        return 0

    lax.fori_loop(0, nb // 16, zbody, 0)


def _hist_fill(ce_v, lab_v, hcnt, hsum, shift, dmask, pshift, pref):
    ones = jnp.ones((16,), jnp.int32)

    def body(i, _):
        off = i * 16
        v = ce_v[pl.ds(off, 16)]
        l = lab_v[pl.ds(off, 16)]
        u = plsc.bitcast(v, jnp.int32)
        msk = (l == 0) & ((u >> pshift) == pref)
        d = (u >> shift) & dmask
        plsc.addupdate_scatter(hcnt, [d], ones, mask=msk)
        plsc.addupdate_scatter(hsum, [d], v, mask=msk)
        return 0

    lax.fori_loop(0, GRP, body, 0)


def _hist_scan(hcnt, hsum, nb, K, G0, S0):
    iota16 = _IOTA16()
    nchunk = nb // 16

    def sbody(jj, st):
        G, S, b, cb, found = st
        j = nchunk - 1 - jj
        cv = hcnt[pl.ds(j * 16, 16)]
        sv = hsum[pl.ds(j * 16, 16)]
        rcv = lax.rev(cv, (0,))
        rsv = lax.rev(sv, (0,))
        ccum = plsc.cumsum(rcv)
        scum = plsc.cumsum(rsv)
        cross = (G + ccum) >= K
        r = jnp.min(jnp.where(cross, iota16, 16))
        hit = r < 16
        oh = iota16 == r
        cR = jnp.sum(jnp.where(oh, ccum, 0))
        cntR = jnp.sum(jnp.where(oh, rcv, 0))
        sR = jnp.sum(jnp.where(oh, scum, 0.0))
        sumR = jnp.sum(jnp.where(oh, rsv, 0.0))
        tot = jnp.sum(cv)
        stot = jnp.sum(sv)
        nf = found | hit
        G2 = jnp.where(found, G, jnp.where(hit, G + cR - cntR, G + tot))
        S2 = jnp.where(found, S, jnp.where(hit, S + sR - sumR, S + stot))
        b2 = jnp.where(found, b, jnp.where(hit, j * 16 + 15 - r, b))
        cb2 = jnp.where(found, cb, jnp.where(hit, cntR, cb))
        return G2, S2, b2, cb2, nf

    G, S, b, cb, _ = lax.fori_loop(
        0, nchunk, sbody, (G0, S0, jnp.int32(0), jnp.int32(0), False))
    return b, G, S, cb


def _sc_body(ce_hbm, lab_hbm, out_hbm, ce_v, lab_v, hcnt, hsum, outv):
    cax = lax.axis_index("c")
    sax = lax.axis_index("s")
    wid = sax * 2 + cax

    pltpu.sync_copy(ce_hbm.at[wid], ce_v)
    pltpu.sync_copy(lab_hbm.at[wid], lab_v)

    zi = jnp.zeros((16,), jnp.int32)
    zf = jnp.zeros((16,), jnp.float32)

    def p1(i, acc):
        npos, pce, nneg, nsum = acc
        off = i * 16
        v = ce_v[pl.ds(off, 16)]
        l = lab_v[pl.ds(off, 16)]
        pos = l > 0
        neg = l == 0
        return (npos + jnp.where(pos, 1, 0),
                pce + jnp.where(pos, v, 0.0),
                nneg + jnp.where(neg, 1, 0),
                nsum + jnp.where(neg, v, 0.0))

    npos_v, pce_v, nneg_v, nsum_v = lax.fori_loop(0, GRP, p1, (zi, zf, zi, zf))
    num_pos = jnp.sum(npos_v)
    n_neg = jnp.sum(nneg_v)
    pos_ce = jnp.sum(pce_v)
    neg_all = jnp.sum(nsum_v)
    K = num_pos * 3

    def all_branch():
        return neg_all

    def radix_branch():
        G = jnp.int32(0)
        S = jnp.float32(0.0)
        _hist_zero(hcnt, hsum, 2048)
        _hist_fill(ce_v, lab_v, hcnt, hsum, 20, 2047, 31, jnp.int32(0))
        b0, G, S, _ = _hist_scan(hcnt, hsum, 2048, K, G, S)
        _hist_zero(hcnt, hsum, 1024)
        _hist_fill(ce_v, lab_v, hcnt, hsum, 10, 1023, 20, b0)
        b1, G, S, _ = _hist_scan(hcnt, hsum, 1024, K, G, S)
        _hist_zero(hcnt, hsum, 1024)
        pref2 = (b0 << 10) | b1
        _hist_fill(ce_v, lab_v, hcnt, hsum, 0, 1023, 10, pref2)
        b2, G, S, T = _hist_scan(hcnt, hsum, 1024, K, G, S)
        vbits = (pref2 << 10) | b2
        V = lax.bitcast_convert_type(vbits, jnp.float32)
        taken = jnp.clip(K - G, 0, T)
        return S + jnp.where(taken > 0, taken.astype(jnp.float32) * V, 0.0)

    neg_sel = lax.cond(K >= n_neg, all_branch, radix_branch)
    cls_row = pos_ce + neg_sel

    iota16 = _IOTA16()
    outv[...] = jnp.where(iota16 == 0, cls_row,
                          jnp.where(iota16 == 1, num_pos.astype(jnp.float32),
                                    0.0))
    pltpu.sync_copy(outv, out_hbm.at[wid])


def _sc_stage(ce_pad, lab_pad):
    mesh = plsc.VectorSubcoreMesh(core_axis_name="c", subcore_axis_name="s",
                                  num_cores=2, num_subcores=16)
    f = functools.partial(
        pl.kernel,
        out_type=jax.ShapeDtypeStruct((B, 16), jnp.float32),
        mesh=mesh,
        compiler_params=pltpu.CompilerParams(needs_layout_passes=False),
        scratch_types=[
            pltpu.VMEM((PP,), jnp.float32),
            pltpu.VMEM((PP,), jnp.int32),
            pltpu.VMEM((2048,), jnp.int32),
            pltpu.VMEM((2048,), jnp.float32),
            pltpu.VMEM((16,), jnp.float32),
        ],
    )(_sc_body)
    return f(ce_pad, lab_pad)


def kernel(confidence, predicted_locations, labels, gt_locations):
    lab3 = labels[..., None]

    ce3, bbvec = _tc_stage(confidence, lab3, predicted_locations,
                           gt_locations)
    ce_pad = ce3.reshape(B, PP)

    lab_pad = jnp.pad(labels, ((0, 0), (0, PP - P)), constant_values=-1)

    rows = _sc_stage(ce_pad, lab_pad)

    cls_total = jnp.sum(rows[:, 0])
    npos_total = jnp.sum(rows[:, 1])
    bbox_loss = bbvec[0, 0] / npos_total
    cls_loss = cls_total / npos_total
    return (bbox_loss, cls_loss)


# speedup vs baseline: 2.6902x; 2.0143x over previous
import functools

import jax
import jax.numpy as jnp
from jax import lax
from jax.experimental import pallas as pl
from jax.experimental.pallas import tpu as pltpu
from jax.experimental.pallas import tpu_sc as plsc

B, P, C = 32, 8732, 81
PP = 8960
TP = 4480
NPT = PP // TP
GRP = PP // 16
P4 = P * 4


def _tc_body(x_ref, lab_ref, bg_ref, gs_ref):
    pt = pl.program_id(1)
    x = x_ref[0]
    labv = lab_ref[0]

    e = jnp.exp(x)
    ones = jnp.ones((C, 1), jnp.float32)
    s = lax.dot_general(e, ones, (((1,), (0,)), ((), ())),
                        preferred_element_type=jnp.float32)

    row = lax.broadcasted_iota(jnp.int32, (TP, 1), 0)
    xz = jnp.where(pt * TP + row < P, x, 0.0)
    crow = lax.broadcasted_iota(jnp.int32, (C, 1), 0)
    ht = jnp.where(crow == labv, 1.0, 0.0)
    prod = lax.dot_general(ht, xz, (((1,), (0,)), ((), ())),
                           preferred_element_type=jnp.float32)
    dg = (lax.broadcasted_iota(jnp.int32, (C, C), 0)
          == lax.broadcasted_iota(jnp.int32, (C, C), 1))
    sum_cl = jnp.sum(jnp.where(dg, prod, 0.0))
    vrow = jnp.ones((1, TP), jnp.float32)
    sum_c0 = lax.dot_general(vrow, xz, (((1,), (0,)), ((), ())),
                             preferred_element_type=jnp.float32)[0, 0]
    gs_blk = sum_cl - sum_c0

    c0 = x[:, 0:1]
    sc2 = jnp.concatenate([s, c0], axis=1).T
    bg_t = jnp.maximum(jnp.log(sc2[0:1, :]) - sc2[1:2, :], 0.0)
    bg_ref[0, 0, pl.ds(pt * TP, TP)] = bg_t[0]

    lane = lax.broadcasted_iota(jnp.int32, (1, 128), 1)
    gs_ref[0, 0, 0, :] = jnp.where(lane == 0, gs_blk, 0.0)[0]


def _tc_stage(conf, lab3):
    return pl.pallas_call(
        _tc_body,
        grid=(B, NPT),
        in_specs=[
            pl.BlockSpec((1, TP, C), lambda b, p: (b, p, 0)),
            pl.BlockSpec((1, 1, TP), lambda b, p: (b, 0, p)),
        ],
        out_specs=[
            pl.BlockSpec((1, 1, PP), lambda b, p: (b, 0, 0)),
            pl.BlockSpec((1, 1, 1, 128), lambda b, p: (b, p, 0, 0)),
        ],
        out_shape=[
            jax.ShapeDtypeStruct((B, 1, PP), jnp.float32),
            jax.ShapeDtypeStruct((B, NPT, 1, 128), jnp.float32),
        ],
        compiler_params=pltpu.CompilerParams(
            dimension_semantics=("parallel", "parallel")),
    )(conf, lab3)


_IOTA16 = lambda: lax.iota(jnp.int32, 16)


def _hist_zero(hcnt, hsum, nb):
    zi = jnp.zeros((16,), jnp.int32)
    zf = jnp.zeros((16,), jnp.float32)

    def zbody(i, _):
        hcnt[pl.ds(i * 16, 16)] = zi
        hsum[pl.ds(i * 16, 16)] = zf
        return 0

    lax.fori_loop(0, nb // 16, zbody, 0)


def _hist_fill(bg_v, lab_v, hcnt, hsum, shift, dmask, pshift, pref):
    ones = jnp.ones((16,), jnp.int32)

    def body(i, _):
        off = i * 16
        v = bg_v[pl.ds(off, 16)]
        l = lab_v[pl.ds(off, 16)]
        u = plsc.bitcast(v, jnp.int32)
        msk = (l == 0) & ((u >> pshift) == pref)
        d = (u >> shift) & dmask
        plsc.addupdate_scatter(hcnt, [d], ones, mask=msk)
        plsc.addupdate_scatter(hsum, [d], v, mask=msk)
        return 0

    lax.fori_loop(0, GRP, body, 0)


def _hist_scan(hcnt, hsum, nb, K, G0, S0):
    iota16 = _IOTA16()
    nchunk = nb // 16

    def sbody(jj, st):
        G, S, b, cb, found = st
        j = nchunk - 1 - jj
        cv = hcnt[pl.ds(j * 16, 16)]
        sv = hsum[pl.ds(j * 16, 16)]
        rcv = lax.rev(cv, (0,))
        rsv = lax.rev(sv, (0,))
        ccum = plsc.cumsum(rcv)
        scum = plsc.cumsum(rsv)
        cross = (G + ccum) >= K
        r = jnp.min(jnp.where(cross, iota16, 16))
        hit = r < 16
        oh = iota16 == r
        cR = jnp.sum(jnp.where(oh, ccum, 0))
        cntR = jnp.sum(jnp.where(oh, rcv, 0))
        sR = jnp.sum(jnp.where(oh, scum, 0.0))
        sumR = jnp.sum(jnp.where(oh, rsv, 0.0))
        tot = jnp.sum(cv)
        stot = jnp.sum(sv)
        nf = found | hit
        G2 = jnp.where(found, G, jnp.where(hit, G + cR - cntR, G + tot))
        S2 = jnp.where(found, S, jnp.where(hit, S + sR - sumR, S + stot))
        b2 = jnp.where(found, b, jnp.where(hit, j * 16 + 15 - r, b))
        cb2 = jnp.where(found, cb, jnp.where(hit, cntR, cb))
        return G2, S2, b2, cb2, nf

    G, S, b, cb, _ = lax.fori_loop(
        0, nchunk, sbody, (G0, S0, jnp.int32(0), jnp.int32(0), False))
    return b, G, S, cb


def _sc_body(bg_hbm, lab_hbm, pred_hbm, gt_hbm, out_hbm,
             bg_v, lab_v, pred_v, gt_v, hcnt, hsum, outv):
    cax = lax.axis_index("c")
    sax = lax.axis_index("s")
    wid = sax * 2 + cax

    pltpu.sync_copy(bg_hbm.at[wid], bg_v)
    pltpu.sync_copy(lab_hbm.at[wid], lab_v)
    pltpu.sync_copy(pred_hbm.at[wid], pred_v)
    pltpu.sync_copy(gt_hbm.at[wid], gt_v)

    zi = jnp.zeros((16,), jnp.int32)
    zf = jnp.zeros((16,), jnp.float32)

    def p1(i, acc):
        npos, pbg, nneg, nsum = acc
        off = i * 16
        v = bg_v[pl.ds(off, 16)]
        l = lab_v[pl.ds(off, 16)]
        pos = l > 0
        neg = l == 0
        return (npos + jnp.where(pos, 1, 0),
                pbg + jnp.where(pos, v, 0.0),
                nneg + jnp.where(neg, 1, 0),
                nsum + jnp.where(neg, v, 0.0))

    npos_v, pbg_v, nneg_v, nsum_v = lax.fori_loop(0, GRP, p1, (zi, zf, zi, zf))
    num_pos = jnp.sum(npos_v)
    n_neg = jnp.sum(nneg_v)
    pos_bg = jnp.sum(pbg_v)
    neg_all = jnp.sum(nsum_v)
    K = num_pos * 3

    def all_branch():
        return neg_all

    def radix_branch():
        G = jnp.int32(0)
        S = jnp.float32(0.0)
        _hist_zero(hcnt, hsum, 2048)
        _hist_fill(bg_v, lab_v, hcnt, hsum, 20, 2047, 31, jnp.int32(0))
        b0, G, S, _ = _hist_scan(hcnt, hsum, 2048, K, G, S)
        _hist_zero(hcnt, hsum, 1024)
        _hist_fill(bg_v, lab_v, hcnt, hsum, 10, 1023, 20, b0)
        b1, G, S, _ = _hist_scan(hcnt, hsum, 1024, K, G, S)
        _hist_zero(hcnt, hsum, 1024)
        pref2 = (b0 << 10) | b1
        _hist_fill(bg_v, lab_v, hcnt, hsum, 0, 1023, 10, pref2)
        b2, G, S, T = _hist_scan(hcnt, hsum, 1024, K, G, S)
        vbits = (pref2 << 10) | b2
        V = lax.bitcast_convert_type(vbits, jnp.float32)
        taken = jnp.clip(K - G, 0, T)
        return S + jnp.where(taken > 0, taken.astype(jnp.float32) * V, 0.0)

    neg_sel = lax.cond(K >= n_neg, all_branch, radix_branch)
    cls_row = pos_bg + neg_sel

    q4 = _IOTA16() >> 2

    def pbb(i, acc):
        off = i * 16
        pv = pred_v[pl.ds(off, 16)]
        gv = gt_v[pl.ds(off, 16)]
        lg = plsc.load_gather(lab_v, [i * 4 + q4])
        d = pv - gv
        ad = jnp.abs(d)
        sl1 = jnp.where(ad < 1.0, 0.5 * d * d, ad - 0.5)
        return acc + jnp.where(lg > 0, sl1, 0.0)

    bb_v = lax.fori_loop(0, P4 // 16, pbb, zf)
    bb_row = jnp.sum(bb_v)

    iota16 = _IOTA16()
    outv[...] = jnp.where(iota16 == 0, cls_row,
                          jnp.where(iota16 == 1, num_pos.astype(jnp.float32),
                                    jnp.where(iota16 == 2, bb_row, 0.0)))
    pltpu.sync_copy(outv, out_hbm.at[wid])


def _sc_stage(bg_pad, lab_pad, predf, gtf):
    mesh = plsc.VectorSubcoreMesh(core_axis_name="c", subcore_axis_name="s",
                                  num_cores=2, num_subcores=16)
    f = functools.partial(
        pl.kernel,
        out_type=jax.ShapeDtypeStruct((B, 16), jnp.float32),
        mesh=mesh,
        compiler_params=pltpu.CompilerParams(needs_layout_passes=False),
        scratch_types=[
            pltpu.VMEM((PP,), jnp.float32),
            pltpu.VMEM((PP,), jnp.int32),
            pltpu.VMEM((P4,), jnp.float32),
            pltpu.VMEM((P4,), jnp.float32),
            pltpu.VMEM((2048,), jnp.int32),
            pltpu.VMEM((2048,), jnp.float32),
            pltpu.VMEM((16,), jnp.float32),
        ],
    )(_sc_body)
    return f(bg_pad, lab_pad, predf, gtf)


def kernel(confidence, predicted_locations, labels, gt_locations):
    lab3 = labels.reshape(B, 1, P)

    bg3, gs4 = _tc_stage(confidence, lab3)

    bg_pad = bg3.reshape(B, PP)
    lab_pad = jnp.pad(labels, ((0, 0), (0, PP - P)), constant_values=-1)
    predf = predicted_locations.reshape(B, P4)
    gtf = gt_locations.reshape(B, P4)

    rows = _sc_stage(bg_pad, lab_pad, predf, gtf)

    gs_total = jnp.sum(gs4[:, :, 0, 0])
    cls_total = jnp.sum(rows[:, 0]) - gs_total
    npos_total = jnp.sum(rows[:, 1])
    bbox_loss = jnp.sum(rows[:, 2]) / npos_total
    cls_loss = cls_total / npos_total
    return (bbox_loss, cls_loss)
